# baseline (device time: 22317 ns/iter reference)
import jax
import jax.numpy as jnp
from jax import lax
from jax.experimental import pallas as pl
from jax.experimental.pallas import tpu as pltpu

N_DEV = 32
N_STEPS = 5


def kernel(x):
    m_per, n = x.shape

    def body(
        x_ref, out_ref, send_bufs, recv_bufs, send_sems, recv_sems, ack_sems
    ):
        my = lax.axis_index("i")
        ones = jnp.ones((1, n), jnp.float32)

        t = x_ref[:, :]
        h = m_per
        while h > 1:
            h //= 2
            t = t[:h, :] * t[h : 2 * h, :]

        r = t
        e = ones
        a = None

        for s in range(N_STEPS):
            d = 1 << s
            send_bufs[s, :, :] = r

            @pl.when(my + d < N_DEV)
            def _(s=s, d=d):
                send = pltpu.make_async_remote_copy(
                    src_ref=send_bufs.at[s],
                    dst_ref=recv_bufs.at[s],
                    send_sem=send_sems.at[s],
                    recv_sem=recv_sems.at[s],
                    device_id=(my + d,),
                    device_id_type=pl.DeviceIdType.MESH,
                )
                send.start()

            if s == 0:
                a = x_ref[:, :]
                k = 1
                while k < m_per:
                    shifted = jnp.concatenate(
                        [jnp.ones((k, n), jnp.float32), a[: m_per - k, :]],
                        axis=0,
                    )
                    a = a * shifted
                    k *= 2

            @pl.when(my >= d)
            def _(s=s, d=d):
                recv = pltpu.make_async_remote_copy(
                    src_ref=send_bufs.at[s],
                    dst_ref=recv_bufs.at[s],
                    send_sem=send_sems.at[s],
                    recv_sem=recv_sems.at[s],
                    device_id=(my - d,),
                    device_id_type=pl.DeviceIdType.MESH,
                )
                recv.wait_recv()
                pl.semaphore_signal(
                    ack_sems.at[s],
                    inc=1,
                    device_id=(my - d,),
                    device_id_type=pl.DeviceIdType.MESH,
                )

            v = jnp.where(my >= d, recv_bufs[s, :, :], ones)
            e = e * v
            r = r * v

        out_ref[:, :] = a * e

        for s in range(N_STEPS):
            d = 1 << s

            @pl.when(my + d < N_DEV)
            def _(s=s, d=d):
                send = pltpu.make_async_remote_copy(
                    src_ref=send_bufs.at[s],
                    dst_ref=recv_bufs.at[s],
                    send_sem=send_sems.at[s],
                    recv_sem=recv_sems.at[s],
                    device_id=(my + d,),
                    device_id_type=pl.DeviceIdType.MESH,
                )
                send.wait_send()
                pl.semaphore_wait(ack_sems.at[s], 1)

    return pl.pallas_call(
        body,
        out_shape=jax.ShapeDtypeStruct((m_per, n), jnp.float32),
        in_specs=[pl.BlockSpec(memory_space=pltpu.VMEM)],
        out_specs=pl.BlockSpec(memory_space=pltpu.VMEM),
        scratch_shapes=[
            pltpu.VMEM((N_STEPS, 1, n), jnp.float32),
            pltpu.VMEM((N_STEPS, 1, n), jnp.float32),
            pltpu.SemaphoreType.DMA((N_STEPS,)),
            pltpu.SemaphoreType.DMA((N_STEPS,)),
            pltpu.SemaphoreType.REGULAR((N_STEPS,)),
        ],
    )(x)


# device time: 16055 ns/iter; 1.3900x vs baseline; 1.3900x over previous
import jax
import jax.numpy as jnp
from jax import lax
from jax.experimental import pallas as pl
from jax.experimental.pallas import tpu as pltpu

N_DEV = 32
N_STEPS = 5


def kernel(x):
    m_per, n = x.shape

    def body(
        x_ref,
        out_ref,
        send_bufs,
        recv_bufs,
        send_sems,
        recv_sems,
        ack_sems,
        entry_sems,
    ):
        my = lax.axis_index("i")
        ones = jnp.ones((1, n), jnp.float32)

        barrier_sem = pltpu.get_barrier_semaphore()
        for nbr in ((my + 1) % N_DEV, (my + N_DEV - 1) % N_DEV):
            pl.semaphore_signal(
                barrier_sem,
                inc=1,
                device_id=(nbr,),
                device_id_type=pl.DeviceIdType.MESH,
            )
        pl.semaphore_wait(barrier_sem, 2)

        for s in range(N_STEPS):
            d = 1 << s

            @pl.when(my >= d)
            def _(s=s, d=d):
                pl.semaphore_signal(
                    entry_sems.at[s],
                    inc=1,
                    device_id=(my - d,),
                    device_id_type=pl.DeviceIdType.MESH,
                )

        t = x_ref[:, :]
        h = m_per
        while h > 1:
            h //= 2
            t = t[:h, :] * t[h : 2 * h, :]

        r = t
        e = ones
        a = None

        for s in range(N_STEPS):
            d = 1 << s
            send_bufs[s, :, :] = r

            @pl.when(my + d < N_DEV)
            def _(s=s, d=d):
                pl.semaphore_wait(entry_sems.at[s], 1)
                send = pltpu.make_async_remote_copy(
                    src_ref=send_bufs.at[s],
                    dst_ref=recv_bufs.at[s],
                    send_sem=send_sems.at[s],
                    recv_sem=recv_sems.at[s],
                    device_id=(my + d,),
                    device_id_type=pl.DeviceIdType.MESH,
                )
                send.start()

            if s == 0:
                a = x_ref[:, :]
                k = 1
                while k < m_per:
                    shifted = jnp.concatenate(
                        [jnp.ones((k, n), jnp.float32), a[: m_per - k, :]],
                        axis=0,
                    )
                    a = a * shifted
                    k *= 2

            @pl.when(my >= d)
            def _(s=s, d=d):
                recv = pltpu.make_async_remote_copy(
                    src_ref=send_bufs.at[s],
                    dst_ref=recv_bufs.at[s],
                    send_sem=send_sems.at[s],
                    recv_sem=recv_sems.at[s],
                    device_id=(my - d,),
                    device_id_type=pl.DeviceIdType.MESH,
                )
                recv.wait_recv()
                pl.semaphore_signal(
                    ack_sems.at[s],
                    inc=1,
                    device_id=(my - d,),
                    device_id_type=pl.DeviceIdType.MESH,
                )

            v = jnp.where(my >= d, recv_bufs[s, :, :], ones)
            e = e * v
            r = r * v

        out_ref[:, :] = a * e

        for s in range(N_STEPS):
            d = 1 << s

            @pl.when(my + d < N_DEV)
            def _(s=s, d=d):
                send = pltpu.make_async_remote_copy(
                    src_ref=send_bufs.at[s],
                    dst_ref=recv_bufs.at[s],
                    send_sem=send_sems.at[s],
                    recv_sem=recv_sems.at[s],
                    device_id=(my + d,),
                    device_id_type=pl.DeviceIdType.MESH,
                )
                send.wait_send()
                pl.semaphore_wait(ack_sems.at[s], 1)

    return pl.pallas_call(
        body,
        out_shape=jax.ShapeDtypeStruct((m_per, n), jnp.float32),
        in_specs=[pl.BlockSpec(memory_space=pltpu.VMEM)],
        out_specs=pl.BlockSpec(memory_space=pltpu.VMEM),
        scratch_shapes=[
            pltpu.VMEM((N_STEPS, 1, n), jnp.float32),
            pltpu.VMEM((N_STEPS, 1, n), jnp.float32),
            pltpu.SemaphoreType.DMA((N_STEPS,)),
            pltpu.SemaphoreType.DMA((N_STEPS,)),
            pltpu.SemaphoreType.REGULAR((N_STEPS,)),
            pltpu.SemaphoreType.REGULAR((N_STEPS,)),
        ],
        compiler_params=pltpu.CompilerParams(collective_id=0),
    )(x)
